# Initial kernel scaffold; baseline (speedup 1.0000x reference)
import jax
import jax.numpy as jnp
from jax.experimental import pallas as pl


def kernel(user_emb, item_emb, adj_indices, adj_values):
    raise NotImplementedError("write your pallas kernel here")


# speedup vs baseline: 4.4126x; 4.4126x over previous
import functools

import jax
import jax.numpy as jnp
from jax import lax
from jax.experimental import pallas as pl
from jax.experimental.pallas import tpu as pltpu
from jax.experimental.pallas import tpu_sc as plsc

_NC = 2
_NS = 16
_L = 16

_C = 256
_CR = _C // 128
_WB = 128


def _spmm_layer(x, cols2d, rows2d, vals, extras, out_scale, n, d, half, r_pad):
    e_pad = vals.shape[0]
    ep_tile = e_pad // _NS
    nch = ep_tile // _C
    tile_rows = r_pad // _NS
    n_extra = len(extras)
    nd16 = d // _L

    mesh = plsc.VectorSubcoreMesh(core_axis_name="c", subcore_axis_name="s")

    def body(*refs):
        x_ref, cols_ref, rows_ref, vals_ref = refs[:4]
        extra_refs = refs[4:4 + n_extra]
        out_ref = refs[4 + n_extra]
        acc, cb, rb, vb, lrb, g, sem_g, sem_s = refs[5 + n_extra:]

        c = lax.axis_index("c")
        s = lax.axis_index("s")
        rlo = c * half

        zv = jnp.zeros((_L,), jnp.float32)

        def zbody(e, carry):
            for dd in range(nd16):
                g[e, pl.ds(dd * _L, _L)] = zv
            return carry

        lax.fori_loop(0, _C, zbody, 0)
        t0 = s * tile_rows
        nfull = tile_rows // _C
        for q in range(nfull):
            pltpu.sync_copy(g, acc.at[pl.ds(t0 + q * _C, _C)])
        rem = tile_rows - nfull * _C
        if rem:
            pltpu.sync_copy(g.at[pl.ds(0, rem)],
                            acc.at[pl.ds(t0 + nfull * _C, rem)])
        plsc.subcore_barrier()

        ebase0 = s * ep_tile

        def chunk(k, carry):
            eb = ebase0 + k * _C
            cp_c = pltpu.async_copy(cols_ref.at[pl.ds(eb, _C)], cb, sem_g)
            cp_r = pltpu.async_copy(rows_ref.at[pl.ds(eb, _C)], rb, sem_g)
            cp_v = pltpu.async_copy(vals_ref.at[pl.ds(eb, _C)], vb, sem_g)
            cp_c.wait()
            cp_r.wait()
            cp_v.wait()
            gthr = [
                pltpu.async_copy(x_ref.at[cb.at[pl.ds(j * 128, 128)]],
                                 g.at[pl.ds(j * 128, 128)], sem_g)
                for j in range(_CR)
            ]
            for i in range(_C // _L):
                r16 = rb[pl.ds(i * _L, _L)]
                ok = (r16 >= rlo) & (r16 < rlo + half)
                lrb[i // 8, pl.ds((i % 8) * _L, _L)] = \
                    jnp.where(ok, r16 - rlo, -1)
            for cp in gthr:
                cp.wait()

            def sbody(e16, carry):
                base = e16 * _L
                vv = vb[pl.ds(base, _L)]
                for l in range(_L):
                    bv = lax.broadcast(vv[l], (_L,))
                    for dd in range(nd16):
                        sl = pl.ds(dd * _L, _L)
                        g[base + l, sl] = g[base + l, sl] * bv
                return carry

            lax.fori_loop(0, _C // _L, sbody, 0)

            for j in range(_CR):
                pltpu.async_copy(
                    g.at[pl.ds(j * 128, 128)],
                    acc.at[plsc.Indices(lrb.at[j], ignored_value=-1)],
                    sem_s, add=True,
                ).wait()
            return carry

        lax.fori_loop(0, nch, chunk, 0)
        plsc.subcore_barrier()

        wlo = s * tile_rows
        whi = jnp.minimum(wlo + tile_rows, half)
        nwb = -(-tile_rows // _WB)
        sc16 = jnp.full((_L,), out_scale, jnp.float32)
        for q in range(nwb):
            st = jnp.minimum(wlo + q * _WB, whi - _WB)
            pltpu.sync_copy(acc.at[pl.ds(st, _WB)], g.at[pl.ds(0, _WB)])
            for xr in extra_refs:
                pltpu.sync_copy(xr.at[pl.ds(rlo + st, _WB)],
                                g.at[pl.ds(_WB, _WB)])

                def abody(e, carry):
                    for dd in range(nd16):
                        sl = pl.ds(dd * _L, _L)
                        g[e, sl] = g[e, sl] + g[_WB + e, sl]
                    return carry

                lax.fori_loop(0, _WB, abody, 0)
            if out_scale != 1.0:

                def mbody(e, carry):
                    for dd in range(nd16):
                        sl = pl.ds(dd * _L, _L)
                        g[e, sl] = g[e, sl] * sc16
                    return carry

                lax.fori_loop(0, _WB, mbody, 0)
            pltpu.sync_copy(g.at[pl.ds(0, _WB)],
                            out_ref.at[pl.ds(rlo + st, _WB)])

    f = pl.kernel(
        body,
        out_type=jax.ShapeDtypeStruct((n, d), jnp.float32),
        mesh=mesh,
        compiler_params=pltpu.CompilerParams(use_tc_tiling_on_sc=False),
        scratch_types=[
            pltpu.VMEM_SHARED((r_pad, d), jnp.float32),
            pltpu.VMEM((_C,), jnp.int32),
            pltpu.VMEM((_C,), jnp.int32),
            pltpu.VMEM((_C,), jnp.float32),
            pltpu.VMEM((_CR, 128), jnp.int32),
            pltpu.VMEM((_C, d), jnp.float32),
            pltpu.SemaphoreType.DMA,
            pltpu.SemaphoreType.DMA,
        ],
    )
    return f(x, cols2d, rows2d, vals, *extras)


def kernel(user_emb, item_emb, adj_indices, adj_values):
    u = user_emb.shape[0]
    n = u + item_emb.shape[0]
    d = user_emb.shape[1]
    half = n // 2

    e = adj_values.shape[0]
    step = _NS * _C
    e_pad = -(-e // step) * step
    pad = e_pad - e
    rows = adj_indices[0]
    cols = adj_indices[1]
    vals = adj_values
    if pad:
        zi = jnp.zeros((pad,), jnp.int32)
        rows = jnp.concatenate([rows, zi])
        cols = jnp.concatenate([cols, zi])
        vals = jnp.concatenate([vals, jnp.zeros((pad,), jnp.float32)])
    tile_rows = -(-(-(-half // _NS)) // 8) * 8
    r_pad = tile_rows * _NS

    x0 = jnp.concatenate([user_emb, item_emb], axis=0)
    args = (cols, rows, vals)
    y1 = _spmm_layer(x0, *args, [], 1.0, n, d, half, r_pad)
    y2 = _spmm_layer(y1, *args, [], 1.0, n, d, half, r_pad)
    out = _spmm_layer(y2, *args, [y1, y2], 1.0 / 3.0, n, d, half, r_pad)
    return (out[:u], out[u:])


# speedup vs baseline: 6.3339x; 1.4354x over previous
import functools

import jax
import jax.numpy as jnp
from jax import lax
from jax.experimental import pallas as pl
from jax.experimental.pallas import tpu as pltpu
from jax.experimental.pallas import tpu_sc as plsc

_NC = 2
_NS = 16
_L = 16

_STEP = 128
_SPG = 3
_GRP = _SPG * _STEP
_WB = 128


def _spmm_layer(x, cols, rows, vals, extras, out_scale, n, d, half, r_pad):
    e_pad = vals.shape[0]
    ep_tile = e_pad // _NS
    ng = ep_tile // _GRP
    tile_rows = r_pad // _NS
    n_extra = len(extras)
    nd16 = d // _L

    mesh = plsc.VectorSubcoreMesh(core_axis_name="c", subcore_axis_name="s")

    def body(*refs):
        x_ref, cols_ref, rows_ref, vals_ref = refs[:4]
        extra_refs = refs[4:4 + n_extra]
        out_ref = refs[4 + n_extra]
        (acc, cbg, rbg, vbg, lrb, g,
         sem_i, sem_g, sem_s, sem_o) = refs[5 + n_extra:]

        c = lax.axis_index("c")
        s = lax.axis_index("s")
        rlo = c * half
        tb = s * ep_tile

        zv = jnp.zeros((_L,), jnp.float32)

        def zbody(e, carry):
            for dd in range(nd16):
                g[0, e, pl.ds(dd * _L, _L)] = zv
            return carry

        lax.fori_loop(0, _STEP, zbody, 0)
        t0 = s * tile_rows
        zcps = []
        nzfull = tile_rows // _STEP
        for q in range(nzfull):
            zcps.append(pltpu.async_copy(
                g.at[0], acc.at[pl.ds(t0 + q * _STEP, _STEP)], sem_o))
        zrem = tile_rows - nzfull * _STEP
        if zrem:
            zcps.append(pltpu.async_copy(
                g.at[0, pl.ds(0, zrem)],
                acc.at[pl.ds(t0 + nzfull * _STEP, zrem)], sem_o))
        for cp in zcps:
            cp.wait()
        plsc.subcore_barrier()

        def idx_dma(grp, sel):
            eb = tb + grp * _GRP
            return (
                pltpu.make_async_copy(cols_ref.at[pl.ds(eb, _GRP)],
                                      cbg.at[sel], sem_i),
                pltpu.make_async_copy(rows_ref.at[pl.ds(eb, _GRP)],
                                      rbg.at[sel], sem_i),
                pltpu.make_async_copy(vals_ref.at[pl.ds(eb, _GRP)],
                                      vbg.at[sel], sem_i),
            )

        def gather_desc(sel, h, buf):
            return pltpu.make_async_copy(
                x_ref.at[cbg.at[sel, pl.ds(h * _STEP, _STEP)]],
                g.at[buf], sem_g)

        def scatter_desc(sel, h, buf):
            return pltpu.make_async_copy(
                g.at[buf],
                acc.at[plsc.Indices(lrb.at[sel, h], ignored_value=-1)],
                sem_s)

        for cp in idx_dma(0, 0):
            cp.start()
        for cp in idx_dma(0, 0):
            cp.wait()
        gather_desc(0, 0, 0).start()

        def group(gi, carry):
            sel = gi & 1
            nsel = 1 - sel

            @pl.when(gi < ng - 1)
            def _():
                for cp in idx_dma(gi + 1, nsel):
                    cp.start()

            for i in range(_GRP // _L):
                r16 = rbg[sel, pl.ds(i * _L, _L)]
                ok = (r16 >= rlo) & (r16 < rlo + half)
                lrb[sel, i // 8, pl.ds((i % 8) * _L, _L)] = \
                    jnp.where(ok, r16 - rlo, -1)

            for h in range(_SPG):
                nh = (h + 1) % _SPG
                if h == _SPG - 1:
                    scatter_desc(sel, nh, nh).wait()
                else:
                    @pl.when(gi > 0)
                    def _():
                        scatter_desc(nsel, nh, nh).wait()
                gather_desc(sel, h, h).wait()
                if h < _SPG - 1:
                    gather_desc(sel, nh, nh).start()
                else:
                    @pl.when(gi < ng - 1)
                    def _():
                        for cp in idx_dma(gi + 1, nsel):
                            cp.wait()
                        gather_desc(nsel, 0, nh).start()
                @plsc.parallel_loop(0, _STEP // _L, unroll=2)
                def _(p):
                    vv = vbg[sel, pl.ds(h * _STEP + p * _L, _L)]
                    base = p * _L
                    for l in range(_L):
                        bv = lax.broadcast(vv[l], (_L,))
                        for dd in range(nd16):
                            sl = pl.ds(dd * _L, _L)
                            g[h, base + l, sl] = g[h, base + l, sl] * bv
                scatter_desc(sel, h, h).start(add=True)
            return carry

        lax.fori_loop(0, ng, group, 0)
        lsel = (ng - 1) & 1
        t_last = ng * _SPG - 1
        scatter_desc(lsel, (t_last - 1) % _SPG, (t_last - 1) % _SPG).wait()
        scatter_desc(lsel, t_last % _SPG, t_last % _SPG).wait()
        plsc.subcore_barrier()

        wlo = s * tile_rows
        whi = jnp.minimum(wlo + tile_rows, half)
        nwb = -(-tile_rows // _WB)
        sc16 = jnp.full((_L,), out_scale, jnp.float32)

        def wb_start(q):
            st = jnp.minimum(wlo + q * _WB, whi - _WB)
            return st, pltpu.async_copy(acc.at[pl.ds(st, _WB)],
                                        g.at[q % 2], sem_g)

        sts = [None] * nwb
        ins = [None] * nwb
        outs = [None] * nwb
        sts[0], ins[0] = wb_start(0)
        for q in range(nwb):
            p = q % 2
            if q + 1 < nwb:
                if q >= 1:
                    outs[q - 1].wait()
                sts[q + 1], ins[q + 1] = wb_start(q + 1)
            ins[q].wait()
            for xr in extra_refs:
                pltpu.sync_copy(xr.at[pl.ds(rlo + sts[q], _WB)], g.at[2])

                def abody(e, carry):
                    for dd in range(nd16):
                        sl = pl.ds(dd * _L, _L)
                        g[p, e, sl] = g[p, e, sl] + g[2, e, sl]
                    return carry

                lax.fori_loop(0, _WB, abody, 0)
            if out_scale != 1.0:

                def mbody(e, carry):
                    for dd in range(nd16):
                        sl = pl.ds(dd * _L, _L)
                        g[p, e, sl] = g[p, e, sl] * sc16
                    return carry

                lax.fori_loop(0, _WB, mbody, 0)
            outs[q] = pltpu.async_copy(
                g.at[p], out_ref.at[pl.ds(rlo + sts[q], _WB)], sem_o)
        outs[nwb - 2].wait()
        outs[nwb - 1].wait()

    f = pl.kernel(
        body,
        out_type=jax.ShapeDtypeStruct((n, d), jnp.float32),
        mesh=mesh,
        compiler_params=pltpu.CompilerParams(use_tc_tiling_on_sc=False),
        scratch_types=[
            pltpu.VMEM_SHARED((r_pad, d), jnp.float32),
            pltpu.VMEM((2, _GRP), jnp.int32),
            pltpu.VMEM((2, _GRP), jnp.int32),
            pltpu.VMEM((2, _GRP), jnp.float32),
            pltpu.VMEM((2, _SPG, 128), jnp.int32),
            pltpu.VMEM((_SPG, _STEP, d), jnp.float32),
            pltpu.SemaphoreType.DMA,
            pltpu.SemaphoreType.DMA,
            pltpu.SemaphoreType.DMA,
            pltpu.SemaphoreType.DMA,
        ],
    )
    return f(x, cols, rows, vals, *extras)


def kernel(user_emb, item_emb, adj_indices, adj_values):
    u = user_emb.shape[0]
    n = u + item_emb.shape[0]
    d = user_emb.shape[1]
    half = n // 2

    e = adj_values.shape[0]
    step = _NS * _GRP
    e_pad = -(-e // step) * step
    pad = e_pad - e
    rows = adj_indices[0]
    cols = adj_indices[1]
    vals = adj_values
    if pad:
        zi = jnp.zeros((pad,), jnp.int32)
        rows = jnp.concatenate([rows, zi])
        cols = jnp.concatenate([cols, zi])
        vals = jnp.concatenate([vals, jnp.zeros((pad,), jnp.float32)])

    tile_rows = -(-(-(-half // _NS)) // 8) * 8
    r_pad = tile_rows * _NS

    x0 = jnp.concatenate([user_emb, item_emb], axis=0)
    args = (cols, rows, vals)
    y1 = _spmm_layer(x0, *args, [], 1.0, n, d, half, r_pad)
    y2 = _spmm_layer(y1, *args, [], 1.0, n, d, half, r_pad)
    out = _spmm_layer(y2, *args, [y1, y2], 1.0 / 3.0, n, d, half, r_pad)
    return (out[:u], out[u:])


# speedup vs baseline: 7.2736x; 1.1484x over previous
import functools

import jax
import jax.numpy as jnp
from jax import lax
from jax.experimental import pallas as pl
from jax.experimental.pallas import tpu as pltpu
from jax.experimental.pallas import tpu_sc as plsc

_NC = 2
_NS = 16
_L = 16

_STEP = 128
_SPG = 3
_GRP = _SPG * _STEP
_WB = 128


def _spmm_layer(x, cols, rows, vals, extras, out_scale, n, d, half, r_pad):
    e_pad = vals.shape[0]
    ep_tile = e_pad // _NS
    ng = ep_tile // _GRP
    tile_rows = r_pad // _NS
    n_extra = len(extras)
    nd16 = d // _L

    mesh = plsc.VectorSubcoreMesh(core_axis_name="c", subcore_axis_name="s")

    def body(*refs):
        x_ref, cols_ref, rows_ref, vals_ref = refs[:4]
        extra_refs = refs[4:4 + n_extra]
        out_ref = refs[4 + n_extra]
        (acc, cbg, rbg, vbg, lrb, mcb, g,
         sem_i, sem_g, sem_s, sem_o) = refs[5 + n_extra:]

        c = lax.axis_index("c")
        s = lax.axis_index("s")
        rlo = c * half
        tb = s * ep_tile

        zv = jnp.zeros((_L,), jnp.float32)

        def zbody(e, carry):
            for dd in range(nd16):
                g[0, e, pl.ds(dd * _L, _L)] = zv
            return carry

        lax.fori_loop(0, _STEP, zbody, 0)
        t0 = s * tile_rows
        zcps = []
        nzfull = tile_rows // _STEP
        for q in range(nzfull):
            zcps.append(pltpu.async_copy(
                g.at[0], acc.at[pl.ds(t0 + q * _STEP, _STEP)], sem_o))
        zrem = tile_rows - nzfull * _STEP
        if zrem:
            zcps.append(pltpu.async_copy(
                g.at[0, pl.ds(0, zrem)],
                acc.at[pl.ds(t0 + nzfull * _STEP, zrem)], sem_o))
        for cp in zcps:
            cp.wait()
        plsc.subcore_barrier()

        def idx_dma(grp, sel):
            eb = tb + grp * _GRP
            return (
                pltpu.make_async_copy(cols_ref.at[pl.ds(eb, _GRP)],
                                      cbg.at[sel], sem_i),
                pltpu.make_async_copy(rows_ref.at[pl.ds(eb, _GRP)],
                                      rbg.at[sel], sem_i),
                pltpu.make_async_copy(vals_ref.at[pl.ds(eb, _GRP)],
                                      vbg.at[sel], sem_i),
            )

        def gather_desc(sel, h, buf):
            return pltpu.make_async_copy(
                x_ref.at[plsc.Indices(mcb.at[sel, h], ignored_value=-1)],
                g.at[buf], sem_g)

        def compute_masks(sel):
            for i in range(_GRP // _L):
                r16 = rbg[sel, pl.ds(i * _L, _L)]
                c16 = cbg[sel, pl.ds(i * _L, _L)]
                ok = (r16 >= rlo) & (r16 < rlo + half)
                sl = pl.ds((i % 8) * _L, _L)
                lrb[sel, i // 8, sl] = jnp.where(ok, r16 - rlo, -1)
                mcb[sel, i // 8, sl] = jnp.where(ok, c16, -1)

        def scatter_desc(sel, h, buf):
            return pltpu.make_async_copy(
                g.at[buf],
                acc.at[plsc.Indices(lrb.at[sel, h], ignored_value=-1)],
                sem_s)

        for cp in idx_dma(0, 0):
            cp.start()
        for cp in idx_dma(0, 0):
            cp.wait()
        compute_masks(0)
        gather_desc(0, 0, 0).start()

        def group(gi, carry):
            sel = gi & 1
            nsel = 1 - sel

            @pl.when(gi < ng - 1)
            def _():
                for cp in idx_dma(gi + 1, nsel):
                    cp.start()

            for h in range(_SPG):
                nh = (h + 1) % _SPG
                if h == _SPG - 1:
                    scatter_desc(sel, nh, nh).wait()
                else:
                    @pl.when(gi > 0)
                    def _():
                        scatter_desc(nsel, nh, nh).wait()
                gather_desc(sel, h, h).wait()
                if h < _SPG - 1:
                    gather_desc(sel, nh, nh).start()
                else:
                    @pl.when(gi < ng - 1)
                    def _():
                        for cp in idx_dma(gi + 1, nsel):
                            cp.wait()
                        compute_masks(nsel)
                        gather_desc(nsel, 0, nh).start()
                @plsc.parallel_loop(0, _STEP // _L, unroll=2)
                def _(p):
                    vv = vbg[sel, pl.ds(h * _STEP + p * _L, _L)]
                    base = p * _L
                    for l in range(_L):
                        bv = lax.broadcast(vv[l], (_L,))
                        for dd in range(nd16):
                            sl = pl.ds(dd * _L, _L)
                            g[h, base + l, sl] = g[h, base + l, sl] * bv
                scatter_desc(sel, h, h).start(add=True)
            return carry

        lax.fori_loop(0, ng, group, 0)
        lsel = (ng - 1) & 1
        t_last = ng * _SPG - 1
        scatter_desc(lsel, (t_last - 1) % _SPG, (t_last - 1) % _SPG).wait()
        scatter_desc(lsel, t_last % _SPG, t_last % _SPG).wait()
        plsc.subcore_barrier()

        wlo = s * tile_rows
        whi = jnp.minimum(wlo + tile_rows, half)
        nwb = -(-tile_rows // _WB)
        sc16 = jnp.full((_L,), out_scale, jnp.float32)

        def wb_start(q):
            st = jnp.minimum(wlo + q * _WB, whi - _WB)
            return st, pltpu.async_copy(acc.at[pl.ds(st, _WB)],
                                        g.at[q % 2], sem_g)

        sts = [None] * nwb
        ins = [None] * nwb
        outs = [None] * nwb
        sts[0], ins[0] = wb_start(0)
        for q in range(nwb):
            p = q % 2
            if q + 1 < nwb:
                if q >= 1:
                    outs[q - 1].wait()
                sts[q + 1], ins[q + 1] = wb_start(q + 1)
            ins[q].wait()
            for xr in extra_refs:
                pltpu.sync_copy(xr.at[pl.ds(rlo + sts[q], _WB)], g.at[2])

                def abody(e, carry):
                    for dd in range(nd16):
                        sl = pl.ds(dd * _L, _L)
                        g[p, e, sl] = g[p, e, sl] + g[2, e, sl]
                    return carry

                lax.fori_loop(0, _WB, abody, 0)
            if out_scale != 1.0:

                def mbody(e, carry):
                    for dd in range(nd16):
                        sl = pl.ds(dd * _L, _L)
                        g[p, e, sl] = g[p, e, sl] * sc16
                    return carry

                lax.fori_loop(0, _WB, mbody, 0)
            outs[q] = pltpu.async_copy(
                g.at[p], out_ref.at[pl.ds(rlo + sts[q], _WB)], sem_o)
        outs[nwb - 2].wait()
        outs[nwb - 1].wait()

    f = pl.kernel(
        body,
        out_type=jax.ShapeDtypeStruct((n, d), jnp.float32),
        mesh=mesh,
        compiler_params=pltpu.CompilerParams(use_tc_tiling_on_sc=False),
        scratch_types=[
            pltpu.VMEM_SHARED((r_pad, d), jnp.float32),
            pltpu.VMEM((2, _GRP), jnp.int32),
            pltpu.VMEM((2, _GRP), jnp.int32),
            pltpu.VMEM((2, _GRP), jnp.float32),
            pltpu.VMEM((2, _SPG, 128), jnp.int32),
            pltpu.VMEM((2, _SPG, 128), jnp.int32),
            pltpu.VMEM((_SPG, _STEP, d), jnp.float32),
            pltpu.SemaphoreType.DMA,
            pltpu.SemaphoreType.DMA,
            pltpu.SemaphoreType.DMA,
            pltpu.SemaphoreType.DMA,
        ],
    )
    return f(x, cols, rows, vals, *extras)


def kernel(user_emb, item_emb, adj_indices, adj_values):
    u = user_emb.shape[0]
    n = u + item_emb.shape[0]
    d = user_emb.shape[1]
    half = n // 2

    e = adj_values.shape[0]
    step = _NS * _GRP
    e_pad = -(-e // step) * step
    pad = e_pad - e
    rows = adj_indices[0]
    cols = adj_indices[1]
    vals = adj_values
    if pad:
        zi = jnp.zeros((pad,), jnp.int32)
        rows = jnp.concatenate([rows, zi])
        cols = jnp.concatenate([cols, zi])
        vals = jnp.concatenate([vals, jnp.zeros((pad,), jnp.float32)])

    tile_rows = -(-(-(-half // _NS)) // 8) * 8
    r_pad = tile_rows * _NS

    x0 = jnp.concatenate([user_emb, item_emb], axis=0)
    args = (cols, rows, vals)
    y1 = _spmm_layer(x0, *args, [], 1.0, n, d, half, r_pad)
    y2 = _spmm_layer(y1, *args, [], 1.0, n, d, half, r_pad)
    out = _spmm_layer(y2, *args, [y1, y2], 1.0 / 3.0, n, d, half, r_pad)
    return (out[:u], out[u:])


# speedup vs baseline: 9.6304x; 1.3240x over previous
import functools

import jax
import jax.numpy as jnp
from jax import lax
from jax.experimental import pallas as pl
from jax.experimental.pallas import tpu as pltpu
from jax.experimental.pallas import tpu_sc as plsc

_NC = 2
_NS = 16
_L = 16

_STEP = 96
_SPG = 4
_GRP = _SPG * _STEP
_WB = 96


def _spmm_layer(x, cols, rows, vals, extras, out_scale, n, d, half, r_pad):
    e_pad = vals.shape[0]
    ep_tile = e_pad // _NS
    ng = ep_tile // _GRP
    tile_rows = r_pad // _NS
    n_extra = len(extras)
    nd16 = d // _L

    mesh = plsc.VectorSubcoreMesh(core_axis_name="c", subcore_axis_name="s")

    def body(*refs):
        x_ref, cols_ref, rows_ref, vals_ref = refs[:4]
        extra_refs = refs[4:4 + n_extra]
        out_ref = refs[4 + n_extra]
        (acc, cbg, rbg, vbg, lrb, mcb, g,
         sem_i, sem_g, sem_s, sem_o) = refs[5 + n_extra:]

        c = lax.axis_index("c")
        s = lax.axis_index("s")
        rlo = c * half
        tb = s * ep_tile

        zv = jnp.zeros((_L,), jnp.float32)

        def zbody(e, carry):
            for dd in range(nd16):
                g[0, e, pl.ds(dd * _L, _L)] = zv
            return carry

        lax.fori_loop(0, _STEP, zbody, 0)
        t0 = s * tile_rows
        zcps = []
        nzfull = tile_rows // _STEP
        for q in range(nzfull):
            zcps.append(pltpu.async_copy(
                g.at[0], acc.at[pl.ds(t0 + q * _STEP, _STEP)], sem_o))
        zrem = tile_rows - nzfull * _STEP
        if zrem:
            zcps.append(pltpu.async_copy(
                g.at[0, pl.ds(0, zrem)],
                acc.at[pl.ds(t0 + nzfull * _STEP, zrem)], sem_o))
        for cp in zcps:
            cp.wait()
        plsc.subcore_barrier()

        def idx_dma(grp, sel):
            eb = tb + grp * _GRP
            return (
                pltpu.make_async_copy(cols_ref.at[pl.ds(eb, _GRP)],
                                      cbg.at[sel], sem_i),
                pltpu.make_async_copy(rows_ref.at[pl.ds(eb, _GRP)],
                                      rbg.at[sel], sem_i),
                pltpu.make_async_copy(vals_ref.at[pl.ds(eb, _GRP)],
                                      vbg.at[sel], sem_i),
            )

        def gather_desc(sel, h, buf):
            return pltpu.make_async_copy(
                x_ref.at[plsc.Indices(mcb.at[sel, h], ignored_value=-1)],
                g.at[buf], sem_g)

        def compute_masks(sel):
            spl = _STEP // _L
            for i in range(_GRP // _L):
                r16 = rbg[sel, pl.ds(i * _L, _L)]
                c16 = cbg[sel, pl.ds(i * _L, _L)]
                ok = (r16 >= rlo) & (r16 < rlo + half)
                sl = pl.ds((i % spl) * _L, _L)
                lrb[sel, i // spl, sl] = jnp.where(ok, r16 - rlo, -1)
                mcb[sel, i // spl, sl] = jnp.where(ok, c16, -1)

        def scatter_desc(sel, h, buf):
            return pltpu.make_async_copy(
                g.at[buf],
                acc.at[plsc.Indices(lrb.at[sel, h], ignored_value=-1)],
                sem_s)

        for cp in idx_dma(0, 0):
            cp.start()
        for cp in idx_dma(0, 0):
            cp.wait()
        compute_masks(0)
        gather_desc(0, 0, 0).start()
        gather_desc(0, 1, 1).start()

        def group(gi, carry):
            sel = gi & 1
            nsel = 1 - sel

            @pl.when(gi < ng - 1)
            def _():
                for cp in idx_dma(gi + 1, nsel):
                    cp.start()

            for h in range(_SPG):
                f2 = (h + 2) % _SPG
                if h >= 2:
                    scatter_desc(sel, f2, f2).wait()
                else:
                    @pl.when(gi > 0)
                    def _():
                        scatter_desc(nsel, f2, f2).wait()
                gather_desc(sel, h, h).wait()
                if h < 2:
                    gather_desc(sel, h + 2, f2).start()
                elif h == 2:
                    @pl.when(gi < ng - 1)
                    def _():
                        for cp in idx_dma(gi + 1, nsel):
                            cp.wait()
                        compute_masks(nsel)
                        gather_desc(nsel, 0, f2).start()
                else:
                    @pl.when(gi < ng - 1)
                    def _():
                        gather_desc(nsel, 1, f2).start()
                @plsc.parallel_loop(0, _STEP // _L, unroll=2)
                def _(p):
                    vv = vbg[sel, pl.ds(h * _STEP + p * _L, _L)]
                    base = p * _L
                    for l in range(_L):
                        bv = lax.broadcast(vv[l], (_L,))
                        for dd in range(nd16):
                            sl = pl.ds(dd * _L, _L)
                            g[h, base + l, sl] = g[h, base + l, sl] * bv
                scatter_desc(sel, h, h).start(add=True)
            return carry

        lax.fori_loop(0, ng, group, 0)
        lsel = (ng - 1) & 1
        t_total = ng * _SPG
        for tt in (t_total - 2, t_total - 1):
            scatter_desc(lsel, tt % _SPG, tt % _SPG).wait()
        plsc.subcore_barrier()

        wlo = s * tile_rows
        whi = jnp.minimum(wlo + tile_rows, half)
        nwb = -(-tile_rows // _WB)
        sc16 = jnp.full((_L,), out_scale, jnp.float32)

        def wb_start(q):
            st = jnp.minimum(wlo + q * _WB, whi - _WB)
            return st, pltpu.async_copy(acc.at[pl.ds(st, _WB)],
                                        g.at[q % 2], sem_g)

        sts = [None] * nwb
        ins = [None] * nwb
        outs = [None] * nwb
        sts[0], ins[0] = wb_start(0)
        for q in range(nwb):
            p = q % 2
            if q + 1 < nwb:
                if q >= 1:
                    outs[q - 1].wait()
                sts[q + 1], ins[q + 1] = wb_start(q + 1)
            ins[q].wait()
            for xr in extra_refs:
                pltpu.sync_copy(xr.at[pl.ds(rlo + sts[q], _WB)], g.at[2])

                def abody(e, carry):
                    for dd in range(nd16):
                        sl = pl.ds(dd * _L, _L)
                        g[p, e, sl] = g[p, e, sl] + g[2, e, sl]
                    return carry

                lax.fori_loop(0, _WB, abody, 0)
            if out_scale != 1.0:

                def mbody(e, carry):
                    for dd in range(nd16):
                        sl = pl.ds(dd * _L, _L)
                        g[p, e, sl] = g[p, e, sl] * sc16
                    return carry

                lax.fori_loop(0, _WB, mbody, 0)
            outs[q] = pltpu.async_copy(
                g.at[p], out_ref.at[pl.ds(rlo + sts[q], _WB)], sem_o)
        outs[nwb - 2].wait()
        outs[nwb - 1].wait()

    f = pl.kernel(
        body,
        out_type=jax.ShapeDtypeStruct((n, d), jnp.float32),
        mesh=mesh,
        compiler_params=pltpu.CompilerParams(use_tc_tiling_on_sc=False),
        scratch_types=[
            pltpu.VMEM_SHARED((r_pad, d), jnp.float32),
            pltpu.VMEM((2, _GRP), jnp.int32),
            pltpu.VMEM((2, _GRP), jnp.int32),
            pltpu.VMEM((2, _GRP), jnp.float32),
            pltpu.VMEM((2, _SPG, _STEP), jnp.int32),
            pltpu.VMEM((2, _SPG, _STEP), jnp.int32),
            pltpu.VMEM((_SPG, _STEP, d), jnp.float32),
            pltpu.SemaphoreType.DMA,
            pltpu.SemaphoreType.DMA,
            pltpu.SemaphoreType.DMA,
            pltpu.SemaphoreType.DMA,
        ],
    )
    return f(x, cols, rows, vals, *extras)


def kernel(user_emb, item_emb, adj_indices, adj_values):
    u = user_emb.shape[0]
    n = u + item_emb.shape[0]
    d = user_emb.shape[1]
    half = n // 2

    e = adj_values.shape[0]
    step = _NS * _GRP
    e_pad = -(-e // step) * step
    pad = e_pad - e
    rows = adj_indices[0]
    cols = adj_indices[1]
    vals = adj_values
    if pad:
        zi = jnp.zeros((pad,), jnp.int32)
        rows = jnp.concatenate([rows, zi])
        cols = jnp.concatenate([cols, zi])
        vals = jnp.concatenate([vals, jnp.zeros((pad,), jnp.float32)])

    tile_rows = -(-(-(-half // _NS)) // 8) * 8
    r_pad = tile_rows * _NS

    x0 = jnp.concatenate([user_emb, item_emb], axis=0)
    args = (cols, rows, vals)
    y1 = _spmm_layer(x0, *args, [], 1.0, n, d, half, r_pad)
    y2 = _spmm_layer(y1, *args, [], 1.0, n, d, half, r_pad)
    out = _spmm_layer(y2, *args, [y1, y2], 1.0 / 3.0, n, d, half, r_pad)
    return (out[:u], out[u:])


# speedup vs baseline: 10.8998x; 1.1318x over previous
import functools

import jax
import jax.numpy as jnp
from jax import lax
from jax.experimental import pallas as pl
from jax.experimental.pallas import tpu as pltpu
from jax.experimental.pallas import tpu_sc as plsc

_NC = 2
_NS = 16
_L = 16

_STEP = 128
_SPG = 5
_GRP = _SPG * _STEP
_WB = 128


def _spmm_layer(x, cols, rows2d, vals, extras, out_scale, n, d2, r_pad):
    e_pad = vals.shape[0]
    ep_tile = e_pad // _NS
    ng = ep_tile // _GRP
    tile_rows = r_pad // _NS
    n_extra = len(extras)
    nd16 = d2 // _L

    mesh = plsc.VectorSubcoreMesh(core_axis_name="c", subcore_axis_name="s")

    def body(*refs):
        x_ref, cols_ref, rows_ref, vals_ref = refs[:4]
        extra_refs = refs[4:4 + n_extra]
        out_ref = refs[4 + n_extra]
        (acc, cbg, rb3, vbg, g,
         sem_i, sem_g, sem_s, sem_o) = refs[5 + n_extra:]

        c = lax.axis_index("c")
        s = lax.axis_index("s")
        tb = s * ep_tile

        zv = jnp.zeros((_L,), jnp.float32)

        def zbody(e, carry):
            for dd in range(nd16):
                g[0, e, pl.ds(dd * _L, _L)] = zv
            return carry

        lax.fori_loop(0, _STEP, zbody, 0)
        t0 = s * tile_rows
        zcps = []
        nzfull = tile_rows // _STEP
        for q in range(nzfull):
            zcps.append(pltpu.async_copy(
                g.at[0], acc.at[pl.ds(t0 + q * _STEP, _STEP)], sem_o))
        zrem = tile_rows - nzfull * _STEP
        if zrem:
            zcps.append(pltpu.async_copy(
                g.at[0, pl.ds(0, zrem)],
                acc.at[pl.ds(t0 + nzfull * _STEP, zrem)], sem_o))
        for cp in zcps:
            cp.wait()
        plsc.subcore_barrier()

        def idx_dma(grp, sel):
            eb = tb + grp * _GRP
            return (
                pltpu.make_async_copy(cols_ref.at[pl.ds(eb, _GRP)],
                                      cbg.at[sel], sem_i),
                pltpu.make_async_copy(
                    rows_ref.at[pl.ds(eb // _STEP, _SPG)],
                    rb3.at[sel], sem_i),
                pltpu.make_async_copy(vals_ref.at[pl.ds(eb, _GRP)],
                                      vbg.at[sel], sem_i),
            )

        def gather_desc(sel, h, buf):
            return pltpu.make_async_copy(
                x_ref.at[c].at[cbg.at[sel, pl.ds(h * _STEP, _STEP)]],
                g.at[buf], sem_g)

        def scatter_desc(sel, h, buf):
            return pltpu.make_async_copy(
                g.at[buf], acc.at[rb3.at[sel, h]], sem_s)

        for cp in idx_dma(0, 0):
            cp.start()
        for cp in idx_dma(0, 0):
            cp.wait()
        for t in range(_SPG - 2):
            gather_desc(0, t, t).start()

        def group(gi, carry):
            sel = gi & 1
            nsel = 1 - sel

            @pl.when(gi < ng - 1)
            def _():
                for cp in idx_dma(gi + 1, nsel):
                    cp.start()

            for h in range(_SPG):
                fb = (h + _SPG - 2) % _SPG
                if h >= 2:
                    scatter_desc(sel, fb, fb).wait()
                else:
                    @pl.when(gi > 0)
                    def _():
                        scatter_desc(nsel, fb, fb).wait()
                gather_desc(sel, h, h).wait()
                if h < 2:
                    gather_desc(sel, h + _SPG - 2, fb).start()
                elif h == 2:
                    @pl.when(gi < ng - 1)
                    def _():
                        for cp in idx_dma(gi + 1, nsel):
                            cp.wait()
                        gather_desc(nsel, 0, fb).start()
                else:
                    @pl.when(gi < ng - 1)
                    def _():
                        gather_desc(nsel, h - 2, fb).start()
                @plsc.parallel_loop(0, _STEP // _L, unroll=2)
                def _(p):
                    vv = vbg[sel, pl.ds(h * _STEP + p * _L, _L)]
                    base = p * _L
                    for l in range(_L):
                        bv = lax.broadcast(vv[l], (_L,))
                        for dd in range(nd16):
                            sl = pl.ds(dd * _L, _L)
                            g[h, base + l, sl] = g[h, base + l, sl] * bv
                scatter_desc(sel, h, h).start(add=True)
            return carry

        lax.fori_loop(0, ng, group, 0)
        lsel = (ng - 1) & 1
        t_total = ng * _SPG
        for tt in (t_total - 2, t_total - 1):
            scatter_desc(lsel, tt % _SPG, tt % _SPG).wait()
        plsc.subcore_barrier()

        wlo = s * tile_rows
        whi = jnp.minimum(wlo + tile_rows, n)
        nwb = -(-tile_rows // _WB)
        sc16 = jnp.full((_L,), out_scale, jnp.float32)

        def wb_start(q):
            st = jnp.minimum(wlo + q * _WB, whi - _WB)
            return st, pltpu.async_copy(acc.at[pl.ds(st, _WB)],
                                        g.at[q % 2], sem_g)

        sts = [None] * nwb
        ins = [None] * nwb
        outs = [None] * nwb
        sts[0], ins[0] = wb_start(0)
        for q in range(nwb):
            p = q % 2
            if q + 1 < nwb:
                if q >= 1:
                    outs[q - 1].wait()
                sts[q + 1], ins[q + 1] = wb_start(q + 1)
            ins[q].wait()
            for xr in extra_refs:
                pltpu.sync_copy(xr.at[c, pl.ds(sts[q], _WB)], g.at[2])

                def abody(e, carry):
                    for dd in range(nd16):
                        sl = pl.ds(dd * _L, _L)
                        g[p, e, sl] = g[p, e, sl] + g[2, e, sl]
                    return carry

                lax.fori_loop(0, _WB, abody, 0)
            if out_scale != 1.0:

                def mbody(e, carry):
                    for dd in range(nd16):
                        sl = pl.ds(dd * _L, _L)
                        g[p, e, sl] = g[p, e, sl] * sc16
                    return carry

                lax.fori_loop(0, _WB, mbody, 0)
            outs[q] = pltpu.async_copy(
                g.at[p], out_ref.at[c, pl.ds(sts[q], _WB)], sem_o)
        outs[nwb - 2].wait()
        outs[nwb - 1].wait()

    f = pl.kernel(
        body,
        out_type=jax.ShapeDtypeStruct((_NC, n, d2), jnp.float32),
        mesh=mesh,
        compiler_params=pltpu.CompilerParams(use_tc_tiling_on_sc=False),
        scratch_types=[
            pltpu.VMEM_SHARED((r_pad, d2), jnp.float32),
            pltpu.VMEM((2, _GRP), jnp.int32),
            pltpu.VMEM((2, _SPG, _STEP), jnp.int32),
            pltpu.VMEM((2, _GRP), jnp.float32),
            pltpu.VMEM((_SPG, _STEP, d2), jnp.float32),
            pltpu.SemaphoreType.DMA,
            pltpu.SemaphoreType.DMA,
            pltpu.SemaphoreType.DMA,
            pltpu.SemaphoreType.DMA,
        ],
    )
    return f(x, cols, rows2d, vals, *extras)


def kernel(user_emb, item_emb, adj_indices, adj_values):
    u = user_emb.shape[0]
    n = u + item_emb.shape[0]
    d = user_emb.shape[1]
    d2 = d // 2

    e = adj_values.shape[0]
    step = _NS * _GRP
    e_pad = -(-e // step) * step
    pad = e_pad - e
    rows = adj_indices[0]
    cols = adj_indices[1]
    vals = adj_values
    if pad:
        zi = jnp.zeros((pad,), jnp.int32)
        rows = jnp.concatenate([rows, zi])
        cols = jnp.concatenate([cols, zi])
        vals = jnp.concatenate([vals, jnp.zeros((pad,), jnp.float32)])
    rows2d = rows.reshape(-1, _STEP)

    tile_rows = -(-(-(-n // _NS)) // 8) * 8
    r_pad = tile_rows * _NS

    x0 = jnp.concatenate([user_emb, item_emb], axis=0)
    xt = jnp.stack([x0[:, :d2], x0[:, d2:]], axis=0)

    args = (cols, rows2d, vals)
    y1 = _spmm_layer(xt, *args, [], 1.0, n, d2, r_pad)
    y2 = _spmm_layer(y1, *args, [], 1.0, n, d2, r_pad)
    out = _spmm_layer(y2, *args, [y1, y2], 1.0 / 3.0, n, d2, r_pad)
    full = jnp.concatenate([out[0], out[1]], axis=1)
    return (full[:u], full[u:])
